# baseline (speedup 1.0000x reference)
import jax
import jax.numpy as jnp
from jax import lax
from jax.experimental import pallas as pl
from jax.experimental.pallas import tpu as pltpu
from jax.experimental.pallas import tpu_sc as plsc

B = 16384
NF = 26
NN = 13
NR = NF + NN
K = 64

NC, NS = 2, 16
NW = NC * NS
BPW = B // NW
NBB = BPW // 128
FS = 2
NCATC = NBB * (NF // FS)
NNUMC = NBB * NN
IDX_ROWS = NF * NBB
NF_PW = NN * BPW
ROW_RUN = 8 * 128
OUT_STRIDE = 128 * ROW_RUN


def _transpose_block(rows_v, stage4, frel):
    iota = lax.iota(jnp.int32, 16)
    i_frel = jnp.full((16,), frel, jnp.int32)
    i_k8 = [(kc * 16 + iota) // 8 for kc in range(K // 16)]
    i_k0 = iota % 8

    def body(i, carry):
        b0 = i * 4
        for u in range(4):
            i_b = jnp.full((16,), b0 + u, jnp.int32)
            for kc in range(K // 16):
                v = rows_v[frel * 128 + b0 + u, pl.ds(kc * 16, 16)]
                plsc.store_scatter(stage4, [i_frel, i_k8[kc], i_k0, i_b], v)
        return carry

    lax.fori_loop(0, 32, body, 0)


def _sc_body(idx_hbm, nf_hbm, table_hbm, net_hbm, out_hbm,
             idx_v, nf_v, net_v, rows0, rows1, stage0, stage1,
             gs0, gs1, os0, os1):
    wid = lax.axis_index("s") * NC + lax.axis_index("c")
    pltpu.sync_copy(net_hbm, net_v)
    pltpu.sync_copy(idx_hbm.at[:, pl.ds(wid * BPW, BPW)], idx_v)
    pltpu.sync_copy(nf_hbm.at[:, pl.ds(wid * BPW, BPW)], nf_v)
    rows = (rows0, rows1)
    stages = (stage0, stage1)
    gsems = (gs0, gs1)
    osems = (os0, os1)

    def cat_gathers(t, slot):
        bbl = t // (NF // FS)
        fc = t % (NF // FS)
        return [
            pltpu.make_async_copy(
                table_hbm.at[idx_v.at[fc * FS + frel,
                                      pl.ds(bbl * 128, 128)]],
                rows[slot].at[pl.ds(frel * 128, 128), :],
                gsems[slot],
            )
            for frel in range(FS)
        ]

    def cat_out(f0, bbg, stage4, osem):
        return pltpu.make_async_copy(
            stage4.at[:, :, :, pl.ds(0, 128)],
            out_hbm.at[pl.ds(f0, FS), :, bbg, :, :],
            osem,
        )

    for cp in cat_gathers(0, 0):
        cp.start()
    for cp in cat_gathers(1, 1):
        cp.start()

    def cat_chunk(i, slot):
        t = 2 * i + slot
        bbl = t // (NF // FS)
        fc = t % (NF // FS)
        f0 = fc * FS
        bbg = wid * NBB + bbl
        for cp in cat_gathers(t, slot):
            cp.wait()

        @pl.when(i >= 1)
        def _():
            pltpu.make_async_copy(
                out_hbm.at[pl.ds(0, FS), :, 0, :, :],
                stages[slot].at[:, :, :, pl.ds(0, 128)],
                osems[slot]).wait()

        for frel in range(FS):
            _transpose_block(rows[slot], stages[slot], frel)
        cat_out(f0, bbg, stages[slot], osems[slot]).start()

        @pl.when(t + 2 < NCATC)
        def _():
            for cp in cat_gathers(t + 2, slot):
                cp.start()

    def cat_loop(i, carry):
        cat_chunk(i, 0)
        cat_chunk(i, 1)
        return carry

    lax.fori_loop(0, NCATC // 2, cat_loop, 0)
    for slot in range(2):
        pltpu.make_async_copy(
            out_hbm.at[pl.ds(0, FS), :, 0, :, :],
            stages[slot].at[:, :, :, pl.ds(0, 128)],
            osems[slot]).wait()

    def num_chunk(i, slot):
        t = 2 * i + slot
        bbl = t // NN
        j = t % NN
        bbg = wid * NBB + bbl
        stage2 = stages[slot]

        @pl.when(i >= 1)
        def _():
            pltpu.make_async_copy(
                out_hbm.at[0, :, 0, :, :],
                stage2.at[0, :, :, pl.ds(0, 128)], osems[slot]).wait()

        netws = [net_v[pl.ds(j * K + kc * 16, 16)] for kc in range(K // 16)]

        def nbody(b16, carry):
            nfvec = nf_v[j, pl.ds(bbl * 128 + b16 * 16, 16)]
            for k in range(K):
                stage2[0, k // 8, k % 8, pl.ds(b16 * 16, 16)] = (
                    netws[k // 16][k % 16] * nfvec)
            return carry

        lax.fori_loop(0, 8, nbody, 0)
        pltpu.make_async_copy(
            stage2.at[0, :, :, pl.ds(0, 128)],
            out_hbm.at[NF + j, :, bbg, :, :],
            osems[slot]).start()

    def num_loop(i, carry):
        num_chunk(i, 0)
        num_chunk(i, 1)
        return carry

    lax.fori_loop(0, NNUMC // 2, num_loop, 0)
    for slot in range(2):
        pltpu.make_async_copy(
            out_hbm.at[0, :, 0, :, :],
            stages[slot].at[0, :, :, pl.ds(0, 128)], osems[slot]).wait()


def kernel(cat_features, num_features, cat_table, num_embedding):
    idx = cat_features.astype(jnp.int32).T
    nf = num_features.T
    net = num_embedding.T.reshape(NN * K)
    mesh = plsc.VectorSubcoreMesh(core_axis_name="c", subcore_axis_name="s")
    f = pl.kernel(
        _sc_body,
        out_type=jax.ShapeDtypeStruct((NR, 8, B // 128, 8, 128),
                                      jnp.float32),
        mesh=mesh,
        compiler_params=pltpu.CompilerParams(
            use_tc_tiling_on_sc=False, needs_layout_passes=False),
        scratch_types=[
            pltpu.VMEM((NF, BPW), jnp.int32),
            pltpu.VMEM((NN, BPW), jnp.float32),
            pltpu.VMEM((NN * K,), jnp.float32),
            pltpu.VMEM((FS * 128, K), jnp.float32),
            pltpu.VMEM((FS * 128, K), jnp.float32),
            pltpu.VMEM((FS, 8, 8, 130), jnp.float32),
            pltpu.VMEM((FS, 8, 8, 130), jnp.float32),
            pltpu.SemaphoreType.DMA,
            pltpu.SemaphoreType.DMA,
            pltpu.SemaphoreType.DMA,
            pltpu.SemaphoreType.DMA,
        ],
    )
    out5 = f(idx, nf, cat_table, net)
    return out5.transpose(2, 4, 0, 1, 3).reshape(B, NR, K)
